# baseline (speedup 1.0000x reference)
import functools

import jax
import jax.numpy as jnp
from jax import lax
from jax.experimental import pallas as pl
from jax.experimental.pallas import tpu as pltpu
from jax.experimental.pallas import tpu_sc as plsc

N = 10000
NPAD = 10240
E = 320000
DIN = 128
DOUT = 128
CHUNK = 128
EDGES_PER_CORE = E // 2
CPC = 1280
PAD_TAIL = CPC * CHUNK - EDGES_PER_CORE
NS = 16
CPT = CPC // NS
NPAIR = CPT // 2
ROWS_PER_TILE = NPAD // NS
RB = 128


def _mm_body(x_ref, w_ref, o_ref):
    o_ref[...] = jnp.dot(x_ref[...], w_ref[...], preferred_element_type=jnp.float32)


def _matmul(x, W):
    bm = 1000
    return pl.pallas_call(
        _mm_body,
        grid=(N // bm,),
        in_specs=[
            pl.BlockSpec((bm, DIN), lambda i: (i, 0)),
            pl.BlockSpec((DIN, DOUT), lambda i: (0, 0)),
        ],
        out_specs=pl.BlockSpec((bm, DOUT), lambda i: (i, 0)),
        out_shape=jax.ShapeDtypeStruct((N, DOUT), jnp.float32),
    )(x, W)


def _combine_body(p_ref, o_ref):
    o_ref[...] = jnp.maximum(p_ref[0] + p_ref[1], 0.0)


def _combine_relu(partials):
    bm = 1000
    return pl.pallas_call(
        _combine_body,
        grid=(N // bm,),
        in_specs=[pl.BlockSpec((2, bm, DOUT), lambda i: (0, i, 0))],
        out_specs=pl.BlockSpec((bm, DOUT), lambda i: (i, 0)),
        out_shape=jax.ShapeDtypeStruct((N, DOUT), jnp.float32),
    )(partials)


@functools.partial(
    pl.kernel,
    out_type=jax.ShapeDtypeStruct((2, NPAD, DOUT), jnp.float32),
    mesh=plsc.VectorSubcoreMesh(core_axis_name="c", subcore_axis_name="s"),
    scratch_types=[
        pltpu.VMEM((CHUNK,), jnp.int32),
        pltpu.VMEM((CHUNK,), jnp.int32),
        pltpu.VMEM((CHUNK,), jnp.float32),
        pltpu.VMEM((CHUNK, DOUT), jnp.float32),
        pltpu.VMEM((CHUNK,), jnp.int32),
        pltpu.VMEM((CHUNK,), jnp.int32),
        pltpu.VMEM((CHUNK,), jnp.float32),
        pltpu.VMEM((CHUNK, DOUT), jnp.float32),
        pltpu.VMEM_SHARED((NPAD, DOUT), jnp.float32),
        pltpu.SemaphoreType.DMA,
        pltpu.SemaphoreType.DMA,
        pltpu.SemaphoreType.DMA,
        pltpu.SemaphoreType.DMA,
    ],
)
def _sc_aggregate(pre_hbm, src_hbm, dst_hbm, ew_hbm, out_hbm,
                  src_a, dst_a, ew_a, rows_a, src_b, dst_b, ew_b, rows_b,
                  acc, sem_ia, sem_ib, sem_ga, sem_gb):
    c = lax.axis_index("c")
    s = lax.axis_index("s")
    row0 = s * ROWS_PER_TILE
    def _idx_copies(i, sv, dv, wv, sem):
        e0 = (c * CPC + s + i * NS) * CHUNK
        return (
            pltpu.make_async_copy(src_hbm.at[pl.ds(e0, CHUNK)], sv, sem),
            pltpu.make_async_copy(dst_hbm.at[pl.ds(e0, CHUNK)], dv, sem),
            pltpu.make_async_copy(ew_hbm.at[pl.ds(e0, CHUNK)], wv, sem),
        )

    def _scale(wv, rv):
        def body(eg, carry):
            w16 = wv[pl.ds(eg * 16, 16)]
            for k in range(16):
                e = eg * 16 + k
                wk = w16[k]
                for j in range(DOUT // 16):
                    sl = pl.ds(j * 16, 16)
                    rv[e, sl] = rv[e, sl] * wk
            return carry

        lax.fori_loop(0, CHUNK // 16, body, 0)

    def _zero_row(r, carry):
        for j in range(DOUT // 16):
            rows_a[r, pl.ds(j * 16, 16)] = jnp.zeros((16,), jnp.float32)
        return carry

    lax.fori_loop(0, RB, _zero_row, 0)
    for b in range(ROWS_PER_TILE // RB):
        pltpu.sync_copy(rows_a.at[pl.ds(0, RB)],
                        acc.at[pl.ds(row0 + b * RB, RB)])
    plsc.subcore_barrier()

    for cp in _idx_copies(0, src_a, dst_a, ew_a, sem_ia):
        cp.start()
    for cp in _idx_copies(1, src_b, dst_b, ew_b, sem_ib):
        cp.start()

    def _pair(t, carry):
        ia = 2 * t
        for cp in _idx_copies(ia, src_a, dst_a, ew_a, sem_ia):
            cp.wait()
        pltpu.make_async_copy(pre_hbm.at[src_a], rows_a, sem_ga).start()
        for cp in _idx_copies(ia + 1, src_b, dst_b, ew_b, sem_ib):
            cp.wait()
        pltpu.make_async_copy(pre_hbm.at[src_b], rows_b, sem_gb).start()
        pltpu.make_async_copy(pre_hbm.at[src_a], rows_a, sem_ga).wait()
        _scale(ew_a, rows_a)
        pltpu.sync_copy(rows_a, acc.at[dst_a], add=True)

        @pl.when(t < NPAIR - 1)
        def _():
            for cp in _idx_copies(ia + 2, src_a, dst_a, ew_a, sem_ia):
                cp.start()

        pltpu.make_async_copy(pre_hbm.at[src_b], rows_b, sem_gb).wait()
        _scale(ew_b, rows_b)
        pltpu.sync_copy(rows_b, acc.at[dst_b], add=True)

        @pl.when(t < NPAIR - 1)
        def _():
            for cp in _idx_copies(ia + 3, src_b, dst_b, ew_b, sem_ib):
                cp.start()

        return carry

    lax.fori_loop(0, NPAIR, _pair, 0)
    plsc.subcore_barrier()

    pltpu.sync_copy(acc.at[pl.ds(row0, ROWS_PER_TILE)],
                    out_hbm.at[c, pl.ds(row0, ROWS_PER_TILE)])


def _pad_split(a):
    z = jnp.zeros((PAD_TAIL,), a.dtype)
    return jnp.concatenate([a[:EDGES_PER_CORE], z, a[EDGES_PER_CORE:], z])


def kernel(x, edge_index, edge_weight, W):
    pre = _matmul(x, W)
    partials = _sc_aggregate(
        pre,
        _pad_split(edge_index[0]),
        _pad_split(edge_index[1]),
        _pad_split(edge_weight),
    )
    return _combine_relu(partials)


# speedup vs baseline: 1.6858x; 1.6858x over previous
import functools

import jax
import jax.numpy as jnp
from jax import lax
from jax.experimental import pallas as pl
from jax.experimental.pallas import tpu as pltpu
from jax.experimental.pallas import tpu_sc as plsc

N = 10000
NPAD = 10240
E = 320000
DIN = 128
DOUT = 128
CHUNK = 128
EDGES_PER_CORE = E // 2
NUM_CHUNKS = EDGES_PER_CORE // CHUNK
NS = 16
ROWS_PER_TILE = NPAD // NS
RB = 128
CHUNKS_PER_TILE = -(-NUM_CHUNKS // NS)


def _mm_body(x_ref, w_ref, o_ref):
    o_ref[...] = jnp.dot(x_ref[...], w_ref[...], preferred_element_type=jnp.float32)


def _matmul(x, W):
    bm = 1000
    return pl.pallas_call(
        _mm_body,
        grid=(N // bm,),
        in_specs=[
            pl.BlockSpec((bm, DIN), lambda i: (i, 0)),
            pl.BlockSpec((DIN, DOUT), lambda i: (0, 0)),
        ],
        out_specs=pl.BlockSpec((bm, DOUT), lambda i: (i, 0)),
        out_shape=jax.ShapeDtypeStruct((N, DOUT), jnp.float32),
    )(x, W)


def _combine_body(p_ref, o_ref):
    o_ref[...] = jnp.maximum(p_ref[0] + p_ref[1], 0.0)


def _combine_relu(partials):
    bm = 1000
    return pl.pallas_call(
        _combine_body,
        grid=(N // bm,),
        in_specs=[pl.BlockSpec((2, bm, DOUT), lambda i: (0, i, 0))],
        out_specs=pl.BlockSpec((bm, DOUT), lambda i: (i, 0)),
        out_shape=jax.ShapeDtypeStruct((N, DOUT), jnp.float32),
    )(partials)


@functools.partial(
    pl.kernel,
    out_type=jax.ShapeDtypeStruct((2, NPAD, DOUT), jnp.float32),
    mesh=plsc.VectorSubcoreMesh(core_axis_name="c", subcore_axis_name="s"),
    scratch_types=[
        pltpu.VMEM((CHUNK,), jnp.int32),
        pltpu.VMEM((CHUNK,), jnp.int32),
        pltpu.VMEM((CHUNK,), jnp.float32),
        pltpu.VMEM((CHUNK, DOUT), jnp.float32),
        pltpu.VMEM_SHARED((NPAD, DOUT), jnp.float32),
        pltpu.SemaphoreType.DMA,
        pltpu.SemaphoreType.DMA,
    ],
)
def _sc_aggregate(pre_hbm, src_hbm, dst_hbm, ew_hbm, out_hbm,
                  src_v, dst_v, ew_v, rows_v, acc, sem_i, sem_g):
    c = lax.axis_index("c")
    s = lax.axis_index("s")
    row0 = s * ROWS_PER_TILE

    def _zero_row(r, carry):
        for j in range(DOUT // 16):
            rows_v[r, pl.ds(j * 16, 16)] = jnp.zeros((16,), jnp.float32)
        return carry

    lax.fori_loop(0, RB, _zero_row, 0)
    for b in range(ROWS_PER_TILE // RB):
        pltpu.sync_copy(rows_v.at[pl.ds(0, RB)],
                        acc.at[pl.ds(row0 + b * RB, RB)])
    plsc.subcore_barrier()

    def _chunk(i, carry):
        g = s + i * NS

        @pl.when(g < NUM_CHUNKS)
        def _():
            e0 = c * EDGES_PER_CORE + g * CHUNK
            cps = (
                pltpu.make_async_copy(src_hbm.at[pl.ds(e0, CHUNK)], src_v, sem_i),
                pltpu.make_async_copy(dst_hbm.at[pl.ds(e0, CHUNK)], dst_v, sem_i),
                pltpu.make_async_copy(ew_hbm.at[pl.ds(e0, CHUNK)], ew_v, sem_i),
            )
            for cp in cps:
                cp.start()
            for cp in cps:
                cp.wait()
            pltpu.async_copy(pre_hbm.at[src_v], rows_v, sem_g).wait()

            def _scale(eg, carry2):
                w16 = ew_v[pl.ds(eg * 16, 16)]
                for k in range(16):
                    e = eg * 16 + k
                    wk = w16[k]
                    for j in range(DOUT // 16):
                        sl = pl.ds(j * 16, 16)
                        rows_v[e, sl] = rows_v[e, sl] * wk
                return carry2

            lax.fori_loop(0, CHUNK // 16, _scale, 0)
            pltpu.sync_copy(rows_v, acc.at[dst_v], add=True)

        return carry

    lax.fori_loop(0, CHUNKS_PER_TILE, _chunk, 0)
    plsc.subcore_barrier()

    pltpu.sync_copy(acc.at[pl.ds(row0, ROWS_PER_TILE)],
                    out_hbm.at[c, pl.ds(row0, ROWS_PER_TILE)])


def kernel(x, edge_index, edge_weight, W):
    pre = _matmul(x, W)
    partials = _sc_aggregate(pre, edge_index[0], edge_index[1], edge_weight)
    return _combine_relu(partials)


# speedup vs baseline: 1.9213x; 1.1397x over previous
import functools

import jax
import jax.numpy as jnp
from jax import lax
from jax.experimental import pallas as pl
from jax.experimental.pallas import tpu as pltpu
from jax.experimental.pallas import tpu_sc as plsc

N = 10000
NPAD = 10240
E = 320000
DIN = 128
DOUT = 128
CHUNK = 128
EDGES_PER_CORE = E // 2
NUM_CHUNKS = EDGES_PER_CORE // CHUNK
NS = 16
ROWS_PER_TILE = NPAD // NS
RB = 128
CHUNKS_PER_TILE = -(-NUM_CHUNKS // NS)


def _mm_body(x_ref, w_ref, o_ref):
    o_ref[...] = jnp.dot(x_ref[...], w_ref[...], preferred_element_type=jnp.float32)


def _matmul(x, W):
    bm = 1000
    return pl.pallas_call(
        _mm_body,
        grid=(N // bm,),
        in_specs=[
            pl.BlockSpec((bm, DIN), lambda i: (i, 0)),
            pl.BlockSpec((DIN, DOUT), lambda i: (0, 0)),
        ],
        out_specs=pl.BlockSpec((bm, DOUT), lambda i: (i, 0)),
        out_shape=jax.ShapeDtypeStruct((N, DOUT), jnp.float32),
    )(x, W)


def _combine_body(p_ref, o_ref):
    o_ref[...] = jnp.maximum(p_ref[0] + p_ref[1], 0.0)


def _combine_relu(partials):
    bm = 1000
    return pl.pallas_call(
        _combine_body,
        grid=(N // bm,),
        in_specs=[pl.BlockSpec((2, bm, DOUT), lambda i: (0, i, 0))],
        out_specs=pl.BlockSpec((bm, DOUT), lambda i: (i, 0)),
        out_shape=jax.ShapeDtypeStruct((N, DOUT), jnp.float32),
    )(partials)


@functools.partial(
    pl.kernel,
    out_type=jax.ShapeDtypeStruct((2, NPAD, DOUT), jnp.float32),
    mesh=plsc.VectorSubcoreMesh(core_axis_name="c", subcore_axis_name="s"),
    scratch_types=[
        pltpu.VMEM((CHUNK,), jnp.int32),
        pltpu.VMEM((CHUNK,), jnp.int32),
        pltpu.VMEM((CHUNK,), jnp.float32),
        pltpu.VMEM((CHUNK,), jnp.int32),
        pltpu.VMEM((CHUNK,), jnp.int32),
        pltpu.VMEM((CHUNK,), jnp.float32),
        pltpu.VMEM((CHUNK, DOUT), jnp.float32),
        pltpu.VMEM_SHARED((NPAD, DOUT), jnp.float32),
        pltpu.SemaphoreType.DMA,
        pltpu.SemaphoreType.DMA,
        pltpu.SemaphoreType.DMA,
    ],
)
def _sc_aggregate(pre_hbm, src_hbm, dst_hbm, ew_hbm, out_hbm,
                  src_a, dst_a, ew_a, src_b, dst_b, ew_b,
                  rows_v, acc, sem_ia, sem_ib, sem_g):
    c = lax.axis_index("c")
    s = lax.axis_index("s")
    row0 = s * ROWS_PER_TILE

    def _idx_copies(i, sv, dv, wv, sem):
        g = s + i * NS
        e0 = c * EDGES_PER_CORE + g * CHUNK
        return (
            pltpu.make_async_copy(src_hbm.at[pl.ds(e0, CHUNK)], sv, sem),
            pltpu.make_async_copy(dst_hbm.at[pl.ds(e0, CHUNK)], dv, sem),
            pltpu.make_async_copy(ew_hbm.at[pl.ds(e0, CHUNK)], wv, sem),
        )

    def _start_idx(i, sv, dv, wv, sem):
        @pl.when(s + i * NS < NUM_CHUNKS)
        def _():
            for cp in _idx_copies(i, sv, dv, wv, sem):
                cp.start()

    def _scale(wv):
        def body(eg, carry2):
            w16 = wv[pl.ds(eg * 16, 16)]
            for k in range(16):
                e = eg * 16 + k
                wk = w16[k]
                for j in range(DOUT // 16):
                    sl = pl.ds(j * 16, 16)
                    rows_v[e, sl] = rows_v[e, sl] * wk
            return carry2

        lax.fori_loop(0, CHUNK // 16, body, 0)

    def _zero_row(r, carry):
        for j in range(DOUT // 16):
            rows_v[r, pl.ds(j * 16, 16)] = jnp.zeros((16,), jnp.float32)
        return carry

    lax.fori_loop(0, RB, _zero_row, 0)
    for b in range(ROWS_PER_TILE // RB):
        pltpu.sync_copy(rows_v.at[pl.ds(0, RB)],
                        acc.at[pl.ds(row0 + b * RB, RB)])
    plsc.subcore_barrier()

    _start_idx(0, src_a, dst_a, ew_a, sem_ia)

    def _slot(i, sv, dv, wv, sem, nsv, ndv, nwv, nsem):
        @pl.when(s + i * NS < NUM_CHUNKS)
        def _():
            for cp in _idx_copies(i, sv, dv, wv, sem):
                cp.wait()
            pltpu.make_async_copy(pre_hbm.at[sv], rows_v, sem_g).start()
            _start_idx(i + 1, nsv, ndv, nwv, nsem)
            pltpu.make_async_copy(pre_hbm.at[sv], rows_v, sem_g).wait()
            _scale(wv)
            pltpu.sync_copy(rows_v, acc.at[dv], add=True)

    def _pair(t, carry):
        _slot(2 * t, src_a, dst_a, ew_a, sem_ia, src_b, dst_b, ew_b, sem_ib)
        _slot(2 * t + 1, src_b, dst_b, ew_b, sem_ib, src_a, dst_a, ew_a, sem_ia)
        return carry

    lax.fori_loop(0, (CHUNKS_PER_TILE + 1) // 2, _pair, 0)
    plsc.subcore_barrier()

    pltpu.sync_copy(acc.at[pl.ds(row0, ROWS_PER_TILE)],
                    out_hbm.at[c, pl.ds(row0, ROWS_PER_TILE)])


def kernel(x, edge_index, edge_weight, W):
    pre = _matmul(x, W)
    partials = _sc_aggregate(pre, edge_index[0], edge_index[1], edge_weight)
    return _combine_relu(partials)


# speedup vs baseline: 2.5117x; 1.3073x over previous
import functools

import jax
import jax.numpy as jnp
from jax import lax
from jax.experimental import pallas as pl
from jax.experimental.pallas import tpu as pltpu
from jax.experimental.pallas import tpu_sc as plsc

N = 10000
NPAD = 10240
E = 320000
DIN = 128
DOUT = 128
CHUNK = 128
EDGES_PER_CORE = E // 2
NUM_CHUNKS = EDGES_PER_CORE // CHUNK
NS = 16
ROWS_PER_TILE = NPAD // NS
RB = 128
CHUNKS_PER_TILE = -(-NUM_CHUNKS // NS)


def _mm_body(x_ref, w_ref, o_ref):
    o_ref[...] = jnp.dot(x_ref[...], w_ref[...], preferred_element_type=jnp.float32)


def _matmul(x, W):
    bm = 1000
    return pl.pallas_call(
        _mm_body,
        grid=(N // bm,),
        in_specs=[
            pl.BlockSpec((bm, DIN), lambda i: (i, 0)),
            pl.BlockSpec((DIN, DOUT), lambda i: (0, 0)),
        ],
        out_specs=pl.BlockSpec((bm, DOUT), lambda i: (i, 0)),
        out_shape=jax.ShapeDtypeStruct((N, DOUT), jnp.float32),
    )(x, W)


def _combine_body(p_ref, o_ref):
    o_ref[...] = jnp.maximum(p_ref[0] + p_ref[1], 0.0)


def _combine_relu(partials):
    bm = 1000
    return pl.pallas_call(
        _combine_body,
        grid=(N // bm,),
        in_specs=[pl.BlockSpec((2, bm, DOUT), lambda i: (0, i, 0))],
        out_specs=pl.BlockSpec((bm, DOUT), lambda i: (i, 0)),
        out_shape=jax.ShapeDtypeStruct((N, DOUT), jnp.float32),
    )(partials)


@functools.partial(
    pl.kernel,
    out_type=jax.ShapeDtypeStruct((2, NPAD, DOUT), jnp.float32),
    mesh=plsc.VectorSubcoreMesh(core_axis_name="c", subcore_axis_name="s"),
    scratch_types=[
        pltpu.VMEM((CHUNK,), jnp.int32),
        pltpu.VMEM((CHUNK,), jnp.int32),
        pltpu.VMEM((CHUNK,), jnp.float32),
        pltpu.VMEM((CHUNK,), jnp.int32),
        pltpu.VMEM((CHUNK,), jnp.int32),
        pltpu.VMEM((CHUNK,), jnp.float32),
        pltpu.VMEM((CHUNK, DOUT), jnp.float32),
        pltpu.VMEM((CHUNK, DOUT), jnp.float32),
        pltpu.VMEM_SHARED((NPAD, DOUT), jnp.float32),
        pltpu.SemaphoreType.DMA,
        pltpu.SemaphoreType.DMA,
        pltpu.SemaphoreType.DMA,
    ],
)
def _sc_aggregate(pre_hbm, src_hbm, dst_hbm, ew_hbm, out_hbm,
                  src_a, dst_a, ew_a, src_b, dst_b, ew_b,
                  rows_a, rows_b, acc, sem_ia, sem_ib, sem_g):
    c = lax.axis_index("c")
    s = lax.axis_index("s")
    row0 = s * ROWS_PER_TILE

    def _idx_copies(i, sv, dv, wv, sem):
        g = s + i * NS
        e0 = c * EDGES_PER_CORE + g * CHUNK
        return (
            pltpu.make_async_copy(src_hbm.at[pl.ds(e0, CHUNK)], sv, sem),
            pltpu.make_async_copy(dst_hbm.at[pl.ds(e0, CHUNK)], dv, sem),
            pltpu.make_async_copy(ew_hbm.at[pl.ds(e0, CHUNK)], wv, sem),
        )

    def _start_idx(i, sv, dv, wv, sem):
        @pl.when(s + i * NS < NUM_CHUNKS)
        def _():
            for cp in _idx_copies(i, sv, dv, wv, sem):
                cp.start()

    def _scale(wv, rv):
        def body(eg, carry2):
            w16 = wv[pl.ds(eg * 16, 16)]
            for k in range(16):
                e = eg * 16 + k
                wk = w16[k]
                for j in range(DOUT // 16):
                    sl = pl.ds(j * 16, 16)
                    rv[e, sl] = rv[e, sl] * wk
            return carry2

        lax.fori_loop(0, CHUNK // 16, body, 0)

    def _zero_row(r, carry):
        for j in range(DOUT // 16):
            rows_a[r, pl.ds(j * 16, 16)] = jnp.zeros((16,), jnp.float32)
        return carry

    lax.fori_loop(0, RB, _zero_row, 0)
    for b in range(ROWS_PER_TILE // RB):
        pltpu.sync_copy(rows_a.at[pl.ds(0, RB)],
                        acc.at[pl.ds(row0 + b * RB, RB)])
    plsc.subcore_barrier()

    _start_idx(0, src_a, dst_a, ew_a, sem_ia)
    _start_idx(1, src_b, dst_b, ew_b, sem_ib)

    @pl.when(s < NUM_CHUNKS)
    def _():
        for cp in _idx_copies(0, src_a, dst_a, ew_a, sem_ia):
            cp.wait()
        pltpu.make_async_copy(pre_hbm.at[src_a], rows_a, sem_g).start()

    def _slot(i, sv, dv, wv, sem, rv, nsv, ndv, nwv, nsem, nrv):
        @pl.when(s + i * NS < NUM_CHUNKS)
        def _():
            pltpu.make_async_copy(pre_hbm.at[sv], rv, sem_g).wait()

            @pl.when(s + (i + 1) * NS < NUM_CHUNKS)
            def _():
                for cp in _idx_copies(i + 1, nsv, ndv, nwv, nsem):
                    cp.wait()
                pltpu.make_async_copy(pre_hbm.at[nsv], nrv, sem_g).start()

            _scale(wv, rv)
            pltpu.sync_copy(rv, acc.at[dv], add=True)
            _start_idx(i + 2, sv, dv, wv, sem)

    def _pair(t, carry):
        _slot(2 * t, src_a, dst_a, ew_a, sem_ia, rows_a,
              src_b, dst_b, ew_b, sem_ib, rows_b)
        _slot(2 * t + 1, src_b, dst_b, ew_b, sem_ib, rows_b,
              src_a, dst_a, ew_a, sem_ia, rows_a)
        return carry

    lax.fori_loop(0, (CHUNKS_PER_TILE + 1) // 2, _pair, 0)
    plsc.subcore_barrier()

    pltpu.sync_copy(acc.at[pl.ds(row0, ROWS_PER_TILE)],
                    out_hbm.at[c, pl.ds(row0, ROWS_PER_TILE)])


def kernel(x, edge_index, edge_weight, W):
    pre = _matmul(x, W)
    partials = _sc_aggregate(pre, edge_index[0], edge_index[1], edge_weight)
    return _combine_relu(partials)
